# baseline (speedup 1.0000x reference)
import functools

import jax
import jax.numpy as jnp
import numpy as np
from jax import lax
from jax.experimental import pallas as pl
from jax.experimental.pallas import tpu as pltpu
from jax.experimental.pallas import tpu_sc as plsc

_B = 8
_S1 = 3
_N = 128
_C = 128
_CW = _C // 2
_OUT = 128
_R = _B * _S1 * _N
_T = _N

_NC = 2
_NS = 16
_NW = _NC * _NS
_RPW = _R // _NW
_NBUF = 4
_LANES = 16

_PERM = np.concatenate(
    [np.concatenate([np.arange(0, 32, 2), np.arange(1, 32, 2)]) + 32 * cb
     for cb in range(_C // 32)]
)


def _row_accumulate(gb, w_v, out_v, r):
    nblk = _CW // _LANES

    def tchunk(tc, accs):
        t0 = tc * _LANES
        w16 = w_v[r, pl.ds(t0, _LANES)]
        accs = list(accs)
        for l in range(_LANES):
            wsp = w16.at[jnp.full((_LANES,), l, jnp.int32)].get(
                mode="promise_in_bounds"
            )
            for cb in range(nblk):
                u = gb[t0 + l, pl.ds(cb * _LANES, _LANES)]
                ab = plsc.bitcast(u, jnp.bfloat16)
                pe, po = plsc.unpack(ab, format=plsc.PackFormat.INTERLEAVED)
                accs[2 * cb] = accs[2 * cb] + wsp * pe
                accs[2 * cb + 1] = accs[2 * cb + 1] + wsp * po
        return tuple(accs)

    zero = tuple(jnp.zeros((_LANES,), jnp.float32) for _ in range(2 * nblk))
    accs = lax.fori_loop(0, _T // _LANES, tchunk, zero)
    for cb in range(nblk):
        out_v[r, pl.ds(cb * 32, _LANES)] = accs[2 * cb]
        out_v[r, pl.ds(cb * 32 + 16, _LANES)] = accs[2 * cb + 1]


@functools.partial(
    pl.kernel,
    mesh=plsc.VectorSubcoreMesh(core_axis_name="c", subcore_axis_name="s"),
    out_type=jax.ShapeDtypeStruct((_R, _C), jnp.float32),
    compiler_params=pltpu.CompilerParams(
        needs_layout_passes=False, use_tc_tiling_on_sc=False
    ),
    scratch_types=(
        [
            pltpu.VMEM((_RPW, _T), jnp.int32),
            pltpu.VMEM((_RPW, _T), jnp.float32),
            pltpu.VMEM((_RPW, _C), jnp.float32),
        ]
        + [pltpu.VMEM((_T, _CW), jnp.int32) for _ in range(_NBUF)]
        + [pltpu.SemaphoreType.DMA for _ in range(_NBUF)]
    ),
)
def _sc_gather_reduce(idx_hbm, w_hbm, x_hbm, out_hbm, idx_v, w_v, out_v, *rest):
    gbufs = rest[:_NBUF]
    sems = rest[_NBUF:]
    wid = lax.axis_index("s") * _NC + lax.axis_index("c")
    base = wid * _RPW

    pltpu.sync_copy(idx_hbm.at[pl.ds(base, _RPW)], idx_v)
    pltpu.sync_copy(w_hbm.at[pl.ds(base, _RPW)], w_v)

    def gather(r, b):
        return pltpu.make_async_copy(x_hbm.at[idx_v.at[r]], gbufs[b], sems[b])

    for b in range(_NBUF):
        gather(b, b).start()

    def outer(i, carry):
        r0 = i * _NBUF
        for b in range(_NBUF):
            r = r0 + b
            gather(r, b).wait()
            _row_accumulate(gbufs[b], w_v, out_v, r)
            nxt = r + _NBUF

            @pl.when(nxt < _RPW)
            def _():
                gather(nxt, b).start()

        return carry

    lax.fori_loop(0, _RPW // _NBUF, outer, 0)
    pltpu.sync_copy(out_v, out_hbm.at[pl.ds(base, _RPW)])


def _dense_body(cum_ref, pw_ref, pb_ref, mw_ref, mb_ref, out_ref):
    acc = None
    for s in range(_S1):
        p = lax.dot_general(
            cum_ref[0, s],
            pw_ref[s],
            (((1,), (1,)), ((), ())),
            preferred_element_type=jnp.float32,
        )
        p = p + pb_ref[s][None, :]
        o = lax.dot_general(
            p,
            mw_ref[:, s * _C:(s + 1) * _C],
            (((1,), (1,)), ((), ())),
            preferred_element_type=jnp.float32,
        )
        acc = o if acc is None else acc + o
    out_ref[0] = acc + mb_ref[0][None, :]


def _tc_dense(cumsum, proj_W, proj_b, mlp_W, mlp_b2d):
    return pl.pallas_call(
        _dense_body,
        grid=(_B,),
        in_specs=[
            pl.BlockSpec((1, _S1, _N, _C), lambda b: (b, 0, 0, 0)),
            pl.BlockSpec((_S1, _C, _C), lambda b: (0, 0, 0)),
            pl.BlockSpec((_S1, _C), lambda b: (0, 0)),
            pl.BlockSpec((_OUT, _S1 * _C), lambda b: (0, 0)),
            pl.BlockSpec((1, _OUT), lambda b: (0, 0)),
        ],
        out_specs=pl.BlockSpec((1, _N, _OUT), lambda b: (b, 0, 0)),
        out_shape=jax.ShapeDtypeStruct((_B, _N, _OUT), jnp.float32),
    )(cumsum, proj_W, proj_b, mlp_W, mlp_b2d)


def kernel(A_binary, A_powers, A_lookup, A_last_edge, x, proj_W, proj_b, mlp_W, mlp_b):
    mask = A_last_edge != 0
    idx = jnp.maximum(A_last_edge.astype(jnp.int32) - 1, 0).reshape(_R, _T)
    w = (A_powers[:, 1:, :, :] * mask.astype(x.dtype)).reshape(_R, _T)
    x_packed = lax.bitcast_convert_type(
        x.astype(jnp.bfloat16).reshape(x.shape[0], _CW, 2), jnp.int32
    )
    cumsum = _sc_gather_reduce(idx, w, x_packed).reshape(_B, _S1, _N, _C)
    proj_Wp = proj_W[:, :, _PERM]
    return _tc_dense(cumsum, proj_Wp, proj_b, mlp_W, mlp_b.reshape(1, _OUT))


# speedup vs baseline: 4.0136x; 4.0136x over previous
import functools

import jax
import jax.numpy as jnp
import numpy as np
from jax import lax
from jax.experimental import pallas as pl
from jax.experimental.pallas import tpu as pltpu
from jax.experimental.pallas import tpu_sc as plsc

_B = 8
_S1 = 3
_N = 128
_C = 128
_CW = _C // 2
_OUT = 128
_R = _B * _S1 * _N
_T = _N

_NC = 2
_NS = 16
_NW = _NC * _NS
_RPW = _R // _NW
_NBUF = 4
_LANES = 16

_PERM = np.concatenate(
    [np.concatenate([np.arange(0, 32, 2), np.arange(1, 32, 2)]) + 32 * cb
     for cb in range(_C // 32)]
)


def _row_accumulate(gb, w_v, out_v, r):
    nblk = _C // _LANES

    def tchunk(tc, accs):
        t0 = tc * _LANES
        w16 = w_v[r, pl.ds(t0, _LANES)]
        accs = list(accs)
        for l in range(_LANES):
            wsp = w16.at[jnp.full((_LANES,), l, jnp.int32)].get(
                mode="promise_in_bounds"
            )
            for cb in range(nblk):
                g = gb[t0 + l, pl.ds(cb * _LANES, _LANES)]
                accs[cb] = accs[cb] + wsp * g
        return tuple(accs)

    zero = tuple(jnp.zeros((_LANES,), jnp.float32) for _ in range(nblk))
    accs = lax.fori_loop(0, _T // _LANES, tchunk, zero)
    for cb in range(nblk):
        out_v[r, pl.ds(cb * _LANES, _LANES)] = accs[cb]


@functools.partial(
    pl.kernel,
    mesh=plsc.VectorSubcoreMesh(core_axis_name="c", subcore_axis_name="s"),
    out_type=jax.ShapeDtypeStruct((_R, _C), jnp.float32),
    scratch_types=(
        [
            pltpu.VMEM((_RPW, _T), jnp.int32),
            pltpu.VMEM((_RPW, _T), jnp.float32),
            pltpu.VMEM((_RPW, _C), jnp.float32),
        ]
        + [pltpu.VMEM((_T, _C), jnp.float32) for _ in range(_NBUF)]
        + [pltpu.SemaphoreType.DMA for _ in range(_NBUF)]
    ),
)
def _sc_gather_reduce(idx_hbm, w_hbm, x_hbm, out_hbm, idx_v, w_v, out_v, *rest):
    gbufs = rest[:_NBUF]
    sems = rest[_NBUF:]
    wid = lax.axis_index("s") * _NC + lax.axis_index("c")
    base = wid * _RPW

    pltpu.sync_copy(idx_hbm.at[pl.ds(base, _RPW)], idx_v)
    pltpu.sync_copy(w_hbm.at[pl.ds(base, _RPW)], w_v)

    def gather(r, b):
        return pltpu.make_async_copy(x_hbm.at[idx_v.at[r]], gbufs[b], sems[b])

    for b in range(_NBUF):
        gather(b, b).start()

    def outer(i, carry):
        r0 = i * _NBUF
        for b in range(_NBUF):
            r = r0 + b
            gather(r, b).wait()
            _row_accumulate(gbufs[b], w_v, out_v, r)
            nxt = r + _NBUF

            @pl.when(nxt < _RPW)
            def _():
                gather(nxt, b).start()

        return carry

    lax.fori_loop(0, _RPW // _NBUF, outer, 0)
    pltpu.sync_copy(out_v, out_hbm.at[pl.ds(base, _RPW)])


def _dense_body(cum_ref, pw_ref, pb_ref, mw_ref, mb_ref, out_ref):
    acc = None
    for s in range(_S1):
        p = lax.dot_general(
            cum_ref[0, s],
            pw_ref[s],
            (((1,), (1,)), ((), ())),
            preferred_element_type=jnp.float32,
        )
        p = p + pb_ref[s][None, :]
        o = lax.dot_general(
            p,
            mw_ref[:, s * _C:(s + 1) * _C],
            (((1,), (1,)), ((), ())),
            preferred_element_type=jnp.float32,
        )
        acc = o if acc is None else acc + o
    out_ref[0] = acc + mb_ref[0][None, :]


def _tc_dense(cumsum, proj_W, proj_b, mlp_W, mlp_b2d):
    return pl.pallas_call(
        _dense_body,
        grid=(_B,),
        in_specs=[
            pl.BlockSpec((1, _S1, _N, _C), lambda b: (b, 0, 0, 0)),
            pl.BlockSpec((_S1, _C, _C), lambda b: (0, 0, 0)),
            pl.BlockSpec((_S1, _C), lambda b: (0, 0)),
            pl.BlockSpec((_OUT, _S1 * _C), lambda b: (0, 0)),
            pl.BlockSpec((1, _OUT), lambda b: (0, 0)),
        ],
        out_specs=pl.BlockSpec((1, _N, _OUT), lambda b: (b, 0, 0)),
        out_shape=jax.ShapeDtypeStruct((_B, _N, _OUT), jnp.float32),
    )(cumsum, proj_W, proj_b, mlp_W, mlp_b2d)


def kernel(A_binary, A_powers, A_lookup, A_last_edge, x, proj_W, proj_b, mlp_W, mlp_b):
    mask = A_last_edge != 0
    idx = jnp.maximum(A_last_edge.astype(jnp.int32) - 1, 0).reshape(_R, _T)
    w = (A_powers[:, 1:, :, :] * mask.astype(x.dtype)).reshape(_R, _T)
    cumsum = _sc_gather_reduce(idx, w, x).reshape(_B, _S1, _N, _C)
    return _tc_dense(cumsum, proj_W, proj_b, mlp_W, mlp_b.reshape(1, _OUT))
